# baseline (speedup 1.0000x reference)
import jax
import jax.numpy as jnp
from jax import lax
from jax.experimental import pallas as pl
from jax.experimental.pallas import tpu as pltpu
from jax.experimental.pallas import tpu_sc as plsc

VOCAB = 100000
SEQ = 2048
BATCH = 4
DIM = 64
B = SEQ * BATCH

_INFO = plsc.get_sparse_core_info()
NC = _INFO.num_cores
NS = _INFO.num_subcores
NW = NC * NS
B_PER_W = B // NW

TR_BK = 2048


def _transpose_body(wt_ref, out_ref):
    row = lax.broadcasted_iota(jnp.int32, (DIM, DIM), 0)
    col = lax.broadcasted_iota(jnp.int32, (DIM, DIM), 1)
    eye = jnp.where(row == col, 1.0, 0.0).astype(jnp.float32)
    out_ref[...] = lax.dot_general(
        wt_ref[...],
        eye,
        dimension_numbers=(((0,), (0,)), ((), ())),
        preferred_element_type=jnp.float32,
        precision=lax.Precision.HIGHEST,
    )


def _transpose_table(Wt):
    grid = (VOCAB + TR_BK - 1) // TR_BK
    return pl.pallas_call(
        _transpose_body,
        grid=(grid,),
        in_specs=[pl.BlockSpec((DIM, TR_BK), lambda i: (0, i))],
        out_specs=pl.BlockSpec((TR_BK, DIM), lambda i: (i, 0)),
        out_shape=jax.ShapeDtypeStruct((VOCAB, DIM), jnp.float32),
    )(Wt)


def _gather_body(idx_hbm, table_hbm, out_hbm, idx_v, rows_v, sem):
    wid = lax.axis_index("s") * NC + lax.axis_index("c")
    base = wid * B_PER_W
    pltpu.sync_copy(idx_hbm.at[pl.ds(base, B_PER_W)], idx_v)

    def fire(g, carry):
        v = idx_v[pl.ds(g * 16, 16)]
        for l in range(16):
            pltpu.make_async_copy(
                table_hbm.at[pl.ds(v[l], 1), :],
                rows_v.at[pl.ds(g * 16 + l, 1), :],
                sem,
            ).start()
        return carry

    lax.fori_loop(0, B_PER_W // 16, fire, 0)

    def drain(j, carry):
        pltpu.make_async_copy(
            table_hbm.at[pl.ds(0, 1), :], rows_v.at[pl.ds(j, 1), :], sem
        ).wait()
        return carry

    lax.fori_loop(0, B_PER_W, drain, 0)
    pltpu.sync_copy(rows_v, out_hbm.at[pl.ds(base, B_PER_W)])


def kernel(input, W):
    idx = input.reshape(B)
    table = _transpose_table(jnp.transpose(W))
    mesh = plsc.VectorSubcoreMesh(core_axis_name="c", subcore_axis_name="s")
    out = pl.kernel(
        _gather_body,
        mesh=mesh,
        out_type=jax.ShapeDtypeStruct((B, DIM), jnp.float32),
        scratch_types=[
            pltpu.VMEM((B_PER_W,), jnp.int32),
            pltpu.VMEM((B_PER_W, DIM), jnp.float32),
            pltpu.SemaphoreType.DMA,
        ],
    )(idx, table)
    return out.reshape(SEQ, BATCH, DIM)


# speedup vs baseline: 1.4358x; 1.4358x over previous
import jax
import jax.numpy as jnp
from jax import lax
from jax.experimental import pallas as pl
from jax.experimental.pallas import tpu as pltpu
from jax.experimental.pallas import tpu_sc as plsc

SEQ = 2048
BATCH = 4
DIM = 64
B = SEQ * BATCH

_INFO = plsc.get_sparse_core_info()
NC = _INFO.num_cores
NS = _INFO.num_subcores
NW = NC * NS
B_PER_W = B // NW


def _gather_body(idx_hbm, table_hbm, out_hbm, idx_v, rows_v, sem):
    wid = lax.axis_index("s") * NC + lax.axis_index("c")
    base = wid * B_PER_W
    pltpu.sync_copy(idx_hbm.at[pl.ds(base, B_PER_W)], idx_v)

    def fire(g, carry):
        v = idx_v[pl.ds(g * 16, 16)]
        for l in range(16):
            pltpu.make_async_copy(
                table_hbm.at[pl.ds(v[l], 1), :],
                rows_v.at[pl.ds(g * 16 + l, 1), :],
                sem,
            ).start()
        return carry

    lax.fori_loop(0, B_PER_W // 16, fire, 0)

    def drain(j, carry):
        pltpu.make_async_copy(
            table_hbm.at[pl.ds(0, 1), :], rows_v.at[pl.ds(j, 1), :], sem
        ).wait()
        return carry

    lax.fori_loop(0, B_PER_W, drain, 0)
    pltpu.sync_copy(rows_v, out_hbm.at[pl.ds(base, B_PER_W)])


def kernel(input, W):
    idx = input.reshape(B)
    mesh = plsc.VectorSubcoreMesh(core_axis_name="c", subcore_axis_name="s")
    out = pl.kernel(
        _gather_body,
        mesh=mesh,
        out_type=jax.ShapeDtypeStruct((B, DIM), jnp.float32),
        scratch_types=[
            pltpu.VMEM((B_PER_W,), jnp.int32),
            pltpu.VMEM((B_PER_W, DIM), jnp.float32),
            pltpu.SemaphoreType.DMA,
        ],
    )(idx, W)
    return out.reshape(SEQ, BATCH, DIM)
